# Initial kernel scaffold; baseline (speedup 1.0000x reference)
import jax
import jax.numpy as jnp
from jax.experimental import pallas as pl


def kernel(vertices, triangles):
    raise NotImplementedError("write your pallas kernel here")


# speedup vs baseline: 2.9170x; 2.9170x over previous
import jax
import jax.numpy as jnp
from jax import lax
from jax.experimental import pallas as pl
from jax.experimental.pallas import tpu as pltpu
from jax.experimental.pallas import tpu_sc as plsc

NUM_VERTICES = 1_000_000
NUM_TRIANGLES = 2_000_000
NUM_IDX = NUM_TRIANGLES * 3
CHUNK = 3200
NCHUNKS = NUM_IDX // CHUNK
NW = 32
PAD = 8


def _gather_body(tri_hbm, verts_hbm, out_hbm, idx_v, rows_v, sem):
    wid = lax.axis_index("s") * 2 + lax.axis_index("c")
    nloc = (NCHUNKS - wid + NW - 1) // NW

    def body(i, _):
        base = (wid + i * NW) * CHUNK
        pltpu.sync_copy(tri_hbm.at[pl.ds(base, CHUNK)], idx_v)
        pltpu.async_copy(verts_hbm.at[idx_v], rows_v, sem).wait()
        pltpu.sync_copy(rows_v, out_hbm.at[pl.ds(base, CHUNK)])
        return 0

    lax.fori_loop(0, nloc, body, 0)


@jax.jit
def _gather(verts_pad, tri_flat):
    mesh = plsc.VectorSubcoreMesh(core_axis_name="c", subcore_axis_name="s")
    fn = pl.kernel(
        _gather_body,
        mesh=mesh,
        compiler_params=pltpu.CompilerParams(use_tc_tiling_on_sc=False),
        out_type=jax.ShapeDtypeStruct((NUM_IDX, PAD), jnp.float32),
        scratch_types=[
            pltpu.VMEM((CHUNK,), jnp.int32),
            pltpu.VMEM((CHUNK, PAD), jnp.float32),
            pltpu.SemaphoreType.DMA,
        ],
    )
    return fn(tri_flat, verts_pad)


def kernel(vertices, triangles):
    tri_flat = triangles.reshape(-1).astype(jnp.int32)
    verts_pad = jnp.pad(vertices, ((0, 0), (0, PAD - 3)))
    out = _gather(verts_pad, tri_flat)
    return out[:, :3].reshape(NUM_TRIANGLES, 3, 3)


# speedup vs baseline: 19.4832x; 6.6791x over previous
import jax
import jax.numpy as jnp
from jax import lax
from jax.experimental import pallas as pl
from jax.experimental.pallas import tpu as pltpu
from jax.experimental.pallas import tpu_sc as plsc

NUM_VERTICES = 1_000_000
NUM_TRIANGLES = 2_000_000
CH = 10000
NCH = NUM_TRIANGLES // CH
NW = 32


def _gather_body(t0, t1, t2, v0, v1, v2, out_hbm, idx_v, row_v, sem_i, sem_g):
    wid = lax.axis_index("s") * 2 + lax.axis_index("c")
    tri_planes = (t0, t1, t2)
    vert_planes = (v0, v1, v2)

    for i in range(3):
        tri = tri_planes[i]

        def body(n, _):
            base = (wid + n * NW) * CH
            pltpu.sync_copy(tri.at[pl.ds(base, CH)], idx_v)
            for k in range(3):
                pltpu.async_copy(vert_planes[k].at[idx_v], row_v, sem_g).wait()
                pltpu.sync_copy(row_v, out_hbm.at[3 * i + k].at[pl.ds(base, CH)])
            return 0

        nloc = (NCH - wid + NW - 1) // NW
        lax.fori_loop(0, nloc, body, 0)


@jax.jit
def _gather(t0, t1, t2, v0, v1, v2):
    mesh = plsc.VectorSubcoreMesh(core_axis_name="c", subcore_axis_name="s")
    fn = pl.kernel(
        _gather_body,
        mesh=mesh,
        compiler_params=pltpu.CompilerParams(use_tc_tiling_on_sc=False),
        out_type=jax.ShapeDtypeStruct((9, NUM_TRIANGLES), jnp.float32),
        scratch_types=[
            pltpu.VMEM((CH,), jnp.int32),
            pltpu.VMEM((CH,), jnp.float32),
            pltpu.SemaphoreType.DMA,
            pltpu.SemaphoreType.DMA,
        ],
    )
    return fn(t0, t1, t2, v0, v1, v2)


def kernel(vertices, triangles):
    tri = triangles.astype(jnp.int32)
    t0, t1, t2 = tri[:, 0], tri[:, 1], tri[:, 2]
    v0, v1, v2 = vertices[:, 0], vertices[:, 1], vertices[:, 2]
    out = _gather(t0, t1, t2, v0, v1, v2)
    return out.reshape(3, 3, NUM_TRIANGLES).transpose(2, 0, 1)
